# baseline (speedup 1.0000x reference)
import functools

import jax
import jax.numpy as jnp
from jax import lax
from jax.experimental import pallas as pl
from jax.experimental.pallas import tpu as pltpu
from jax.experimental.pallas import tpu_sc as plsc

BATCH = 4096
SEQ = 50
SEQ_PAD = 56
VOCAB = 1000
D = 1000

NUM_WORKERS = 32
B_PER_WORKER = BATCH // NUM_WORKERS
NBUF = 2

_MESH = plsc.VectorSubcoreMesh(core_axis_name="c", subcore_axis_name="s")


@functools.partial(
    pl.kernel,
    out_type=jax.ShapeDtypeStruct((BATCH, SEQ, D), jnp.float32),
    mesh=_MESH,
    scratch_types=[
        pltpu.VMEM((B_PER_WORKER, SEQ_PAD), jnp.int32),
        pltpu.VMEM((NBUF, SEQ_PAD, D), jnp.float32),
        pltpu.SemaphoreType.DMA((NBUF,)),
        pltpu.SemaphoreType.DMA((NBUF,)),
    ],
    compiler_params=pltpu.CompilerParams(use_tc_tiling_on_sc=False),
)
def _gather_rows(x_hbm, table_hbm, out_hbm, idx_v, rows, sem_g, sem_w):
    wid = lax.axis_index("s") * 2 + lax.axis_index("c")
    base = wid * B_PER_WORKER

    def gather_copy(k, b):
        return pltpu.make_async_copy(table_hbm.at[idx_v.at[k]], rows.at[b],
                                     sem_g.at[b])

    def write_copy(bb, b):
        return pltpu.make_async_copy(rows.at[b].at[pl.ds(0, SEQ)],
                                     out_hbm.at[bb], sem_w.at[b])

    pltpu.sync_copy(x_hbm.at[pl.ds(base, B_PER_WORKER)], idx_v)

    for b in range(NBUF):
        gather_copy(b, b).start()

    def step(g, cr):
        for b in range(NBUF):
            k = g + b
            gather_copy(k, b).wait()
            write_copy(base + k, b).start()

            @pl.when(k + NBUF < B_PER_WORKER)
            def _():
                write_copy(base + k, b).wait()
                gather_copy(k + NBUF, b).start()

        return cr

    lax.fori_loop(0, B_PER_WORKER // NBUF, lambda i, cr: step(i * NBUF, cr), 0)
    for b in range(NBUF):
        write_copy(base + B_PER_WORKER - NBUF + b, b).wait()


def kernel(x, table):
    x_pad = jnp.pad(x, ((0, 0), (0, SEQ_PAD - SEQ)), mode="edge")
    return _gather_rows(x_pad, table)


# speedup vs baseline: 1.8663x; 1.8663x over previous
import functools

import jax
import jax.numpy as jnp
from jax import lax
from jax.experimental import pallas as pl
from jax.experimental.pallas import tpu as pltpu
from jax.experimental.pallas import tpu_sc as plsc

BATCH = 4096
SEQ = 50
SEQ_PAD = 56
VOCAB = 1000
D = 1000
NSHARD = 8
TAIL = D - 128 * (NSHARD - 1)

NUM_WORKERS = 32
NB = BATCH // NUM_WORKERS
NBUF = 2

_MESH = plsc.VectorSubcoreMesh(core_axis_name="c", subcore_axis_name="s")


@functools.partial(
    pl.kernel,
    out_type=jax.ShapeDtypeStruct((BATCH, SEQ, D), jnp.float32),
    mesh=_MESH,
    scratch_types=[
        pltpu.VMEM((NB * SEQ_PAD,), jnp.int32),
        pltpu.VMEM((NBUF, NSHARD, SEQ, 128), jnp.float32),
        pltpu.VMEM((SEQ, TAIL), jnp.float32),
        pltpu.SemaphoreType.DMA((NBUF,)),
        pltpu.SemaphoreType.DMA((NBUF,)),
        pltpu.SemaphoreType.DMA,
    ],
    compiler_params=pltpu.CompilerParams(use_tc_tiling_on_sc=True),
)
def _gather_rows(xf_hbm, *refs):
    shards = refs[:NSHARD]
    out_hbm = refs[NSHARD]
    idx_v, rows, tail_v, sem_g, sem_w, sem_t = refs[NSHARD + 1:]

    wid = lax.axis_index("s") * 2 + lax.axis_index("c")
    base = wid * NB

    def idx_slice(k):
        return idx_v.at[pl.ds(pl.multiple_of(k * SEQ_PAD, 8), SEQ)]

    def gather_copy(k, m, c):
        return pltpu.make_async_copy(shards[c].at[idx_slice(k)],
                                     rows.at[m, c], sem_g.at[m])

    def shard_write(bb, m, c):
        return pltpu.make_async_copy(
            rows.at[m, c], out_hbm.at[bb].at[:, pl.ds(c * 128, 128)],
            sem_w.at[m])

    def tail_write(bb):
        return pltpu.make_async_copy(
            tail_v, out_hbm.at[bb].at[:, pl.ds(128 * (NSHARD - 1), TAIL)],
            sem_t)

    def tail_compact(m):
        def row(s, cr):
            for off in (0, 16, 32, 48, 64, 80, TAIL - 16):
                tail_v[s, pl.ds(off, 16)] = rows[m, NSHARD - 1, s,
                                                 pl.ds(off, 16)]
            return cr

        lax.fori_loop(0, SEQ, row, 0)

    pltpu.sync_copy(xf_hbm.at[pl.ds(base * SEQ_PAD, NB * SEQ_PAD)], idx_v)

    for m in range(NBUF):
        for c in range(NSHARD):
            gather_copy(m, m, c).start()

    def pair(g, cr):
        for m in range(NBUF):
            k = g + m
            bb = base + k
            for c in range(NSHARD):
                gather_copy(k, m, c).wait()
            for c in range(NSHARD - 1):
                shard_write(bb, m, c).start()

            @pl.when(k >= 1)
            def _():
                tail_write(bb - 1).wait()

            tail_compact(m)
            tail_write(bb).start()

            @pl.when(k + NBUF < NB)
            def _():
                for c in range(NSHARD - 1):
                    shard_write(bb, m, c).wait()
                for c in range(NSHARD):
                    gather_copy(k + NBUF, m, c).start()

        return cr

    lax.fori_loop(0, NB // NBUF, lambda i, cr: pair(i * NBUF, cr), 0)

    for m in range(NBUF):
        for c in range(NSHARD - 1):
            shard_write(base + NB - NBUF + m, m, c).wait()
    tail_write(base + NB - 1).wait()


def kernel(x, table):
    x_flat = jnp.pad(x, ((0, 0), (0, SEQ_PAD - SEQ))).reshape(-1)
    tp = jnp.pad(table, ((0, 0), (0, NSHARD * 128 - D)))
    shards = tuple(tp[:, c * 128:(c + 1) * 128] for c in range(NSHARD))
    return _gather_rows(x_flat, *shards)
